# baseline (speedup 1.0000x reference)
import numpy as np

import jax
import jax.numpy as jnp
from jax import lax
from jax.experimental import pallas as pl
from jax.experimental.pallas import tpu as pltpu
from jax.experimental.pallas import tpu_sc as plsc

N_ROWS = 4800
ROW_LEN = 64
PACKED = 16
L = 16
NTILES = 16
ACTIVE = 15
CHUNK = N_ROWS // ACTIVE
CVECS = CHUNK // L
NEG_INF = float("-inf")
BIG = np.int32(1 << 30)

_GDN = lax.GatherDimensionNumbers(
    offset_dims=(), collapsed_slice_dims=(0,), start_index_map=(0,))


def _dg(v, perm):
    return lax.gather(v, perm[:, None], _GDN, (1,),
                      mode=lax.GatherScatterMode.PROMISE_IN_BOUNDS)


def _lane0(v):
    return jnp.reshape(lax.slice(v, (0,), (1,)), ())


def _lane(v, l, consts):
    if l == 0:
        return _lane0(v)
    return _lane0(_dg(v, consts["xorp"][l]))


def _beats(ak, ai, bk, bi):
    one = jnp.ones((L,), jnp.int32)
    zero = jnp.zeros((L,), jnp.int32)
    tie = jnp.where(ai < bi, one, zero)
    return jnp.where(ak > bk, one, jnp.where(ak == bk, tie, zero))


def _cmpx(k, i, perm, keepw):
    pk = _dg(k, perm)
    pi = _dg(i, perm)
    take = _beats(k, i, pk, pi) == keepw
    return jnp.where(take, k, pk), jnp.where(take, i, pi)


def _sort16(k, i, consts):
    for perm, keepw in consts["sort"]:
        k, i = _cmpx(k, i, perm, keepw)
    return k, i


def _cleanup(k, i, consts):
    for perm, keepw in consts["clean"]:
        k, i = _cmpx(k, i, perm, keepw)
    return k, i


def _merge(rk, ri, bk, bi, consts):
    rbk = lax.rev(bk, (0,))
    rbi = lax.rev(bi, (0,))
    win = _beats(rk, ri, rbk, rbi) != 0
    nk = jnp.where(win, rk, rbk)
    ni = jnp.where(win, ri, rbi)
    return _cleanup(nk, ni, consts)


def _or_reduce0(d, consts):
    for p in consts["bfly"]:
        d = d | _dg(d, p)
    return _lane0(d)


def _make_consts():
    iota = lax.iota(jnp.int32, L)
    c = {}
    c["sort"] = []
    for s in range(1, 5):
        for j in range(s - 1, -1, -1):
            upb = (iota >> j) & 1
            descb = (iota >> s) & 1
            c["sort"].append((iota ^ (1 << j), 1 - (upb ^ descb)))
    c["clean"] = [(iota ^ (1 << j), 1 - ((iota >> j) & 1))
                  for j in (3, 2, 1, 0)]
    c["bfly"] = [iota ^ m for m in (1, 2, 4, 8)]
    c["xorp"] = [iota ^ l for l in range(L)]
    return c


def _sc_body(q_hbm, titles_hbm, cos_hbm, scores_out, titles_out,
             ex_k, ex_i, ex_m,
             qv, tv, sv, stage_k, stage_i, stage_m,
             all_k, all_i, all_m, t16p, sem):
    w = lax.axis_index("s")
    iota = lax.iota(jnp.int32, L)
    consts = _make_consts()

    @pl.when(w < ACTIVE)
    def _():
        pltpu.sync_copy(q_hbm, qv)
        pltpu.sync_copy(
            titles_hbm.at[pl.ds(w * CHUNK * PACKED, CHUNK * PACKED)], tv)
        q = qv[...]

        def row_body(r, cand):
            row = tv[pl.ds(r * PACKED, L)]
            m = row == q
            return jnp.minimum(cand, jnp.where(m, jnp.full((L,), r, jnp.int32),
                                               jnp.full((L,), BIG, jnp.int32)))

        cand = lax.fori_loop(0, CHUNK, row_body,
                             jnp.full((L,), BIG, jnp.int32))
        acc = jnp.int32(-1)
        for l in range(L):
            local = _lane(cand, l, consts)
            lc = jnp.clip(local, 0, CHUNK - 1)
            row = tv[pl.ds(lc * PACKED, L)]
            dd = _or_reduce0(row ^ q, consts)
            hit = (dd == 0) & (local < CHUNK)
            acc = jnp.where(hit, w * CHUNK + local, acc)
        stage_m[...] = jnp.full((L,), acc, jnp.int32)

    @pl.when(w >= ACTIVE)
    def _():
        stage_m[...] = jnp.full((L,), -1, jnp.int32)

    pltpu.sync_copy(stage_m, ex_m.at[w, :])
    plsc.subcore_barrier()

    pltpu.sync_copy(ex_m, all_m)
    mv = all_m[0, :]
    for t in range(1, NTILES):
        mv = jnp.maximum(mv, all_m[t, :])
    idx0 = _lane0(mv)

    @pl.when(idx0 < 0)
    def _():
        @pl.when(w < ACTIVE)
        def _():
            q = qv[...]

            def row_body(r, acc):
                dd = _or_reduce0(tv[pl.ds(r * PACKED, L)] ^ q, consts)
                return jnp.where(dd == 0, w * CHUNK + r, acc)

            acc = lax.fori_loop(0, CHUNK, row_body, jnp.int32(-1))
            stage_m[...] = jnp.full((L,), acc, jnp.int32)

        pltpu.sync_copy(stage_m, ex_m.at[w, :])
        plsc.subcore_barrier()

    pltpu.sync_copy(ex_m, all_m)
    mv = all_m[0, :]
    for t in range(1, NTILES):
        mv = jnp.maximum(mv, all_m[t, :])
    idx = _lane0(mv)

    @pl.when(w < ACTIVE)
    def _():
        pltpu.sync_copy(cos_hbm.at[pl.ds(idx * N_ROWS + w * CHUNK, CHUNK)],
                        sv)

        def topk_body(c, carry):
            rk, ri = carry
            k = sv[pl.ds(c * L, L)]
            ids = iota + (w * CHUNK + c * L)
            sk, si = _sort16(k, ids, consts)
            return _merge(rk, ri, sk, si, consts)

        rk, ri = lax.fori_loop(
            0, CVECS, topk_body,
            (jnp.full((L,), NEG_INF, jnp.float32), jnp.zeros((L,), jnp.int32)))
        stage_k[...] = rk
        stage_i[...] = ri

    @pl.when(w >= ACTIVE)
    def _():
        stage_k[...] = jnp.full((L,), NEG_INF, jnp.float32)
        stage_i[...] = jnp.zeros((L,), jnp.int32)

    pltpu.sync_copy(stage_k, ex_k.at[w, :])
    pltpu.sync_copy(stage_i, ex_i.at[w, :])
    plsc.subcore_barrier()

    @pl.when(w == 0)
    def _():
        pltpu.sync_copy(ex_k, all_k)
        pltpu.sync_copy(ex_i, all_i)
        rk = all_k[0, :]
        ri = all_i[0, :]
        for t in range(1, NTILES):
            rk, ri = _merge(rk, ri, all_k[t, :], all_i[t, :], consts)
        stage_k[...] = rk
        pltpu.sync_copy(stage_k, scores_out)
        copies = []
        for l in range(L):
            rid = _lane(ri, l, consts)
            copies.append(pltpu.async_copy(
                titles_hbm.at[pl.ds(rid * PACKED, PACKED)],
                t16p.at[l, :], sem))
        for cp in copies:
            cp.wait()
        pltpu.sync_copy(t16p, titles_out)


@jax.jit
def kernel(movie_title, original_titles, overview_cos_sim):
    tp = lax.bitcast_convert_type(
        original_titles.astype(jnp.uint8).reshape(N_ROWS, PACKED, 4),
        jnp.int32).reshape(N_ROWS * PACKED)
    qp = lax.bitcast_convert_type(
        movie_title.astype(jnp.uint8).reshape(PACKED, 4), jnp.int32)
    cos_flat = overview_cos_sim.reshape(N_ROWS * N_ROWS)
    mesh = plsc.VectorSubcoreMesh(core_axis_name="c", subcore_axis_name="s",
                                  num_cores=1, num_subcores=NTILES)
    scores16, titles16p, _exk, _exi, _exm = pl.kernel(
        _sc_body,
        out_type=(
            jax.ShapeDtypeStruct((L,), jnp.float32),
            jax.ShapeDtypeStruct((L, PACKED), jnp.int32),
            jax.ShapeDtypeStruct((NTILES, L), jnp.float32),
            jax.ShapeDtypeStruct((NTILES, L), jnp.int32),
            jax.ShapeDtypeStruct((NTILES, L), jnp.int32),
        ),
        mesh=mesh,
        scratch_types=[
            pltpu.VMEM((PACKED,), jnp.int32),
            pltpu.VMEM((CHUNK * PACKED,), jnp.int32),
            pltpu.VMEM((CHUNK,), jnp.float32),
            pltpu.VMEM((L,), jnp.float32),
            pltpu.VMEM((L,), jnp.int32),
            pltpu.VMEM((L,), jnp.int32),
            pltpu.VMEM((NTILES, L), jnp.float32),
            pltpu.VMEM((NTILES, L), jnp.int32),
            pltpu.VMEM((NTILES, L), jnp.int32),
            pltpu.VMEM((L, PACKED), jnp.int32),
            pltpu.SemaphoreType.DMA,
        ],
    )(qp, tp, cos_flat)
    titles = lax.bitcast_convert_type(
        titles16p, jnp.uint8).reshape(L, ROW_LEN).astype(jnp.int32)
    return titles[1:11], scores16[1:11]


# speedup vs baseline: 3.4560x; 3.4560x over previous
import jax
import jax.numpy as jnp
from jax import lax
from jax.experimental import pallas as pl
from jax.experimental.pallas import tpu as pltpu
from jax.experimental.pallas import tpu_sc as plsc

N_ROWS = 4800
ROW_LEN = 64
L = 16
QV = ROW_LEN // L
NTILES = 16
ACTIVE = 15
CHUNK = N_ROWS // ACTIVE
CVECS = CHUNK // L
NEG_INF = float("-inf")
BIG = 1 << 30

_GDN = lax.GatherDimensionNumbers(
    offset_dims=(), collapsed_slice_dims=(0,), start_index_map=(0,))


def _dg(v, perm):
    return lax.gather(v, perm[:, None], _GDN, (1,),
                      mode=lax.GatherScatterMode.PROMISE_IN_BOUNDS)


def _lane0(v):
    return jnp.reshape(lax.slice(v, (0,), (1,)), ())


def _lane(v, l, consts):
    if l == 0:
        return _lane0(v)
    return _lane0(_dg(v, consts["xorp"][l]))


def _beats(ak, ai, bk, bi):
    one = jnp.ones((L,), jnp.int32)
    zero = jnp.zeros((L,), jnp.int32)
    tie = jnp.where(ai < bi, one, zero)
    return jnp.where(ak > bk, one, jnp.where(ak == bk, tie, zero))


def _cmpx(k, i, perm, keepw):
    pk = _dg(k, perm)
    pi = _dg(i, perm)
    take = _beats(k, i, pk, pi) == keepw
    return jnp.where(take, k, pk), jnp.where(take, i, pi)


def _sort16(k, i, consts):
    for perm, keepw in consts["sort"]:
        k, i = _cmpx(k, i, perm, keepw)
    return k, i


def _cleanup(k, i, consts):
    for perm, keepw in consts["clean"]:
        k, i = _cmpx(k, i, perm, keepw)
    return k, i


def _merge(rk, ri, bk, bi, consts):
    rbk = lax.rev(bk, (0,))
    rbi = lax.rev(bi, (0,))
    win = _beats(rk, ri, rbk, rbi) != 0
    nk = jnp.where(win, rk, rbk)
    ni = jnp.where(win, ri, rbi)
    return _cleanup(nk, ni, consts)


def _or_reduce0(d, consts):
    for p in consts["bfly"]:
        d = d | _dg(d, p)
    return _lane0(d)


def _make_consts():
    iota = lax.iota(jnp.int32, L)
    c = {}
    c["sort"] = []
    for s in range(1, 5):
        for j in range(s - 1, -1, -1):
            upb = (iota >> j) & 1
            descb = (iota >> s) & 1
            c["sort"].append((iota ^ (1 << j), 1 - (upb ^ descb)))
    c["clean"] = [(iota ^ (1 << j), 1 - ((iota >> j) & 1))
                  for j in (3, 2, 1, 0)]
    c["bfly"] = [iota ^ m for m in (1, 2, 4, 8)]
    c["xorp"] = [iota ^ l for l in range(L)]
    return c


def _sc_body(q_hbm, titles_hbm, cos_hbm, scores_out, titles_out,
             ex_k, ex_i, ex_m,
             qv, tv, sv8, svp, stage_k, stage_i, stage_m,
             all_k, all_i, all_m, t8s, t16v, sem):
    w = lax.axis_index("s")
    iota = lax.iota(jnp.int32, L)
    consts = _make_consts()

    @pl.when(w < ACTIVE)
    def _():
        pltpu.sync_copy(q_hbm, qv)
        pltpu.sync_copy(titles_hbm.at[pl.ds(w * CHUNK, CHUNK), :], tv)
        qs = [qv[pl.ds(k * L, L)] for k in range(QV)]

        def row_body(r, cand):
            rfull = jnp.full((L,), r, jnp.int32)
            big = jnp.full((L,), BIG, jnp.int32)
            for k in range(QV):
                m = tv[r, pl.ds(k * L, L)] == qs[k]
                cand = jnp.minimum(cand, jnp.where(m, rfull, big))
            return cand

        cand = lax.fori_loop(0, CHUNK, row_body,
                             jnp.full((L,), BIG, jnp.int32))
        acc = jnp.int32(-1)
        for l in range(L):
            local = _lane(cand, l, consts)
            lc = jnp.clip(local, 0, CHUNK - 1)
            d = tv[lc, pl.ds(0, L)] ^ qs[0]
            for k in range(1, QV):
                d = d | (tv[lc, pl.ds(k * L, L)] ^ qs[k])
            dd = _or_reduce0(d, consts)
            hit = (dd == 0) & (local < CHUNK)
            acc = jnp.where(hit, w * CHUNK + local, acc)
        stage_m[...] = jnp.full((L,), acc, jnp.int32)

    @pl.when(w >= ACTIVE)
    def _():
        stage_m[...] = jnp.full((L,), -1, jnp.int32)

    pltpu.sync_copy(stage_m, ex_m.at[w, :])
    plsc.subcore_barrier()

    pltpu.sync_copy(ex_m, all_m)
    mv = all_m[0, :]
    for t in range(1, NTILES):
        mv = jnp.maximum(mv, all_m[t, :])
    idx0 = _lane0(mv)

    @pl.when(idx0 < 0)
    def _():
        @pl.when(w < ACTIVE)
        def _():
            qs = [qv[pl.ds(k * L, L)] for k in range(QV)]

            def row_body(r, acc):
                d = tv[r, pl.ds(0, L)] ^ qs[0]
                for k in range(1, QV):
                    d = d | (tv[r, pl.ds(k * L, L)] ^ qs[k])
                dd = _or_reduce0(d, consts)
                return jnp.where(dd == 0, w * CHUNK + r, acc)

            acc = lax.fori_loop(0, CHUNK, row_body, jnp.int32(-1))
            stage_m[...] = jnp.full((L,), acc, jnp.int32)

        pltpu.sync_copy(stage_m, ex_m.at[w, :])
        plsc.subcore_barrier()

    pltpu.sync_copy(ex_m, all_m)
    mv = all_m[0, :]
    for t in range(1, NTILES):
        mv = jnp.maximum(mv, all_m[t, :])
    idx = _lane0(mv)
    base8 = pl.multiple_of((idx // 8) * 8, 8)
    r8 = idx - base8

    cb = pl.multiple_of(
        jnp.where(w < 8, 384 * w, 3072 + 256 * (w - 8)), 128)
    width = jnp.where(w < 8, 384, jnp.where(w < 14, 256, 192))

    @pl.when(w < ACTIVE)
    def _():
        pltpu.sync_copy(
            cos_hbm.at[pl.ds(base8, 8), pl.ds(cb, 128)], sv8.at[0])

        @pl.when(w < 14)
        def _():
            pltpu.sync_copy(
                cos_hbm.at[pl.ds(base8, 8),
                           pl.ds(pl.multiple_of(cb + 128, 128), 128)],
                sv8.at[1])

        @pl.when(w == 14)
        def _():
            pltpu.sync_copy(
                cos_hbm.at[pl.ds(base8, 8), pl.ds(4736, 64)], svp)
            for j in range(4):
                sv8[1, r8, pl.ds(j * L, L)] = svp[r8, pl.ds(j * L, L)]

        @pl.when(w < 8)
        def _():
            pltpu.sync_copy(
                cos_hbm.at[pl.ds(base8, 8),
                           pl.ds(pl.multiple_of(cb + 256, 128), 128)],
                sv8.at[2])

        limit = cb + width

        def topk_body(c, carry):
            rk, ri = carry
            seg = c // 8
            off = (c % 8) * L
            kraw = sv8[seg, r8, pl.ds(off, L)]
            gid = iota + (cb + seg * 128 + off)
            vmask = gid < limit
            k = jnp.where(vmask, kraw, jnp.full((L,), -1.0, jnp.float32))
            ids = jnp.minimum(gid, N_ROWS - 1)
            sk, si = _sort16(k, ids, consts)
            return _merge(rk, ri, sk, si, consts)

        rk, ri = lax.fori_loop(
            0, 24, topk_body,
            (jnp.full((L,), NEG_INF, jnp.float32), jnp.zeros((L,), jnp.int32)))
        stage_k[...] = rk
        stage_i[...] = ri

    @pl.when(w >= ACTIVE)
    def _():
        stage_k[...] = jnp.full((L,), NEG_INF, jnp.float32)
        stage_i[...] = jnp.zeros((L,), jnp.int32)

    pltpu.sync_copy(stage_k, ex_k.at[w, :])
    pltpu.sync_copy(stage_i, ex_i.at[w, :])
    plsc.subcore_barrier()

    @pl.when(w == 0)
    def _():
        pltpu.sync_copy(ex_k, all_k)
        pltpu.sync_copy(ex_i, all_i)
        rk = all_k[0, :]
        ri = all_i[0, :]
        for t in range(1, NTILES):
            rk, ri = _merge(rk, ri, all_k[t, :], all_i[t, :], consts)
        stage_k[...] = rk
        pltpu.sync_copy(stage_k, scores_out)
        r8s = []
        copies = []
        for l in range(L):
            rid = _lane(ri, l, consts)
            b8 = pl.multiple_of((rid // 8) * 8, 8)
            r8s.append(rid - b8)
            copies.append(pltpu.async_copy(
                titles_hbm.at[pl.ds(b8, 8), :], t8s.at[l], sem))
        for cp in copies:
            cp.wait()
        for l in range(L):
            for k in range(QV):
                t16v[l, pl.ds(k * L, L)] = t8s[l, r8s[l], pl.ds(k * L, L)]
        pltpu.sync_copy(t16v, titles_out)


@jax.jit
def kernel(movie_title, original_titles, overview_cos_sim):
    mesh = plsc.VectorSubcoreMesh(core_axis_name="c", subcore_axis_name="s",
                                  num_cores=1, num_subcores=NTILES)
    scores16, titles16, _exk, _exi, _exm = pl.kernel(
        _sc_body,
        out_type=(
            jax.ShapeDtypeStruct((L,), jnp.float32),
            jax.ShapeDtypeStruct((L, ROW_LEN), jnp.int32),
            jax.ShapeDtypeStruct((NTILES, L), jnp.float32),
            jax.ShapeDtypeStruct((NTILES, L), jnp.int32),
            jax.ShapeDtypeStruct((NTILES, L), jnp.int32),
        ),
        mesh=mesh,
        scratch_types=[
            pltpu.VMEM((ROW_LEN,), jnp.int32),
            pltpu.VMEM((CHUNK, ROW_LEN), jnp.int32),
            pltpu.VMEM((3, 8, 128), jnp.float32),
            pltpu.VMEM((8, 64), jnp.float32),
            pltpu.VMEM((L,), jnp.float32),
            pltpu.VMEM((L,), jnp.int32),
            pltpu.VMEM((L,), jnp.int32),
            pltpu.VMEM((NTILES, L), jnp.float32),
            pltpu.VMEM((NTILES, L), jnp.int32),
            pltpu.VMEM((NTILES, L), jnp.int32),
            pltpu.VMEM((L, 8, ROW_LEN), jnp.int32),
            pltpu.VMEM((L, ROW_LEN), jnp.int32),
            pltpu.SemaphoreType.DMA,
        ],
    )(movie_title, original_titles, overview_cos_sim)
    return titles16[1:11], scores16[1:11]


# speedup vs baseline: 3.6359x; 1.0520x over previous
import jax
import jax.numpy as jnp
from jax import lax
from jax.experimental import pallas as pl
from jax.experimental.pallas import tpu as pltpu
from jax.experimental.pallas import tpu_sc as plsc

N_ROWS = 4800
ROW_LEN = 64
L = 16
QV = ROW_LEN // L
NTILES = 16
ACTIVE = 15
CHUNK = N_ROWS // ACTIVE
CVECS = CHUNK // L
NEG_INF = float("-inf")
BIG = 1 << 30

_GDN = lax.GatherDimensionNumbers(
    offset_dims=(), collapsed_slice_dims=(0,), start_index_map=(0,))


def _dg(v, perm):
    return lax.gather(v, perm[:, None], _GDN, (1,),
                      mode=lax.GatherScatterMode.PROMISE_IN_BOUNDS)


def _lane0(v):
    return jnp.reshape(lax.slice(v, (0,), (1,)), ())


def _lane(v, l, consts):
    if l == 0:
        return _lane0(v)
    return _lane0(_dg(v, consts["xorp"][l]))


def _beats(ak, ai, bk, bi):
    one = jnp.ones((L,), jnp.int32)
    zero = jnp.zeros((L,), jnp.int32)
    tie = jnp.where(ai < bi, one, zero)
    return jnp.where(ak > bk, one, jnp.where(ak == bk, tie, zero))


def _cmpx(k, i, perm, keepw):
    pk = _dg(k, perm)
    pi = _dg(i, perm)
    take = _beats(k, i, pk, pi) == keepw
    return jnp.where(take, k, pk), jnp.where(take, i, pi)


def _sort16(k, i, consts):
    for perm, keepw in consts["sort"]:
        k, i = _cmpx(k, i, perm, keepw)
    return k, i


def _cleanup(k, i, consts):
    for perm, keepw in consts["clean"]:
        k, i = _cmpx(k, i, perm, keepw)
    return k, i


def _merge(rk, ri, bk, bi, consts):
    rbk = lax.rev(bk, (0,))
    rbi = lax.rev(bi, (0,))
    win = _beats(rk, ri, rbk, rbi) != 0
    nk = jnp.where(win, rk, rbk)
    ni = jnp.where(win, ri, rbi)
    return _cleanup(nk, ni, consts)


def _or_reduce0(d, consts):
    for p in consts["bfly"]:
        d = d | _dg(d, p)
    return _lane0(d)


def _make_consts():
    iota = lax.iota(jnp.int32, L)
    c = {}
    c["sort"] = []
    for s in range(1, 5):
        for j in range(s - 1, -1, -1):
            upb = (iota >> j) & 1
            descb = (iota >> s) & 1
            c["sort"].append((iota ^ (1 << j), 1 - (upb ^ descb)))
    c["clean"] = [(iota ^ (1 << j), 1 - ((iota >> j) & 1))
                  for j in (3, 2, 1, 0)]
    c["bfly"] = [iota ^ m for m in (1, 2, 4, 8)]
    c["xorp"] = [iota ^ l for l in range(L)]
    return c


def _sc_body(q_hbm, titles_hbm, cos_hbm, scores_out, titles_out,
             ex_k, ex_i, ex_m,
             qv, tv, sv8, svp, stage_k, stage_i, stage_m,
             all_k, all_i, all_m, t8s, t16v, sm, sem):
    w = lax.axis_index("s")
    iota = lax.iota(jnp.int32, L)
    consts = _make_consts()

    @pl.when(w < ACTIVE)
    def _():
        pltpu.sync_copy(q_hbm, qv)
        pltpu.sync_copy(titles_hbm.at[pl.ds(w * CHUNK, CHUNK), :], tv)
        qs = [qv[pl.ds(k * L, L)] for k in range(QV)]

        def row_body(i, cand):
            big = jnp.full((L,), BIG, jnp.int32)
            for u in range(4):
                r = i * 4 + u
                d = (tv[r, pl.ds(0, L)] ^ qs[0]) | (tv[r, pl.ds(L, L)] ^ qs[1])
                m = d == 0
                cand = jnp.minimum(
                    cand, jnp.where(m, jnp.full((L,), r, jnp.int32), big))
            return cand

        cand = lax.fori_loop(0, CHUNK // 4, row_body,
                             jnp.full((L,), BIG, jnp.int32))
        acc = jnp.int32(-1)
        for l in range(L):
            local = _lane(cand, l, consts)
            lc = jnp.clip(local, 0, CHUNK - 1)
            d = tv[lc, pl.ds(0, L)] ^ qs[0]
            for k in range(1, QV):
                d = d | (tv[lc, pl.ds(k * L, L)] ^ qs[k])
            dd = _or_reduce0(d, consts)
            hit = (dd == 0) & (local < CHUNK)
            acc = jnp.where(hit, w * CHUNK + local, acc)
        stage_m[...] = jnp.full((L,), acc, jnp.int32)

    @pl.when(w >= ACTIVE)
    def _():
        stage_m[...] = jnp.full((L,), -1, jnp.int32)

    pltpu.sync_copy(stage_m, ex_m.at[w, :])
    plsc.subcore_barrier()

    pltpu.sync_copy(ex_m, all_m)
    mv = all_m[0, :]
    for t in range(1, NTILES):
        mv = jnp.maximum(mv, all_m[t, :])
    idx0 = _lane0(mv)
    sm[0] = idx0

    @pl.when(idx0 < 0)
    def _():
        @pl.when(w < ACTIVE)
        def _():
            qs = [qv[pl.ds(k * L, L)] for k in range(QV)]

            def row_body(r, acc):
                d = tv[r, pl.ds(0, L)] ^ qs[0]
                for k in range(1, QV):
                    d = d | (tv[r, pl.ds(k * L, L)] ^ qs[k])
                dd = _or_reduce0(d, consts)
                return jnp.where(dd == 0, w * CHUNK + r, acc)

            acc = lax.fori_loop(0, CHUNK, row_body, jnp.int32(-1))
            stage_m[...] = jnp.full((L,), acc, jnp.int32)

        pltpu.sync_copy(stage_m, ex_m.at[w, :])
        plsc.subcore_barrier()
        pltpu.sync_copy(ex_m, all_m)
        mv2 = all_m[0, :]
        for t in range(1, NTILES):
            mv2 = jnp.maximum(mv2, all_m[t, :])
        sm[0] = _lane0(mv2)

    idx = sm[0]
    base8 = pl.multiple_of((idx // 8) * 8, 8)
    r8 = idx - base8

    cb = pl.multiple_of(
        jnp.where(w < 8, 384 * w, 3072 + 256 * (w - 8)), 128)
    width = jnp.where(w < 8, 384, jnp.where(w < 14, 256, 192))

    @pl.when(w < ACTIVE)
    def _():
        pltpu.sync_copy(
            cos_hbm.at[pl.ds(base8, 8), pl.ds(cb, 128)], sv8.at[0])

        @pl.when(w < 14)
        def _():
            pltpu.sync_copy(
                cos_hbm.at[pl.ds(base8, 8),
                           pl.ds(pl.multiple_of(cb + 128, 128), 128)],
                sv8.at[1])

        @pl.when(w == 14)
        def _():
            pltpu.sync_copy(
                cos_hbm.at[pl.ds(base8, 8), pl.ds(4736, 64)], svp)
            for j in range(4):
                sv8[1, r8, pl.ds(j * L, L)] = svp[r8, pl.ds(j * L, L)]

        @pl.when(w < 8)
        def _():
            pltpu.sync_copy(
                cos_hbm.at[pl.ds(base8, 8),
                           pl.ds(pl.multiple_of(cb + 256, 128), 128)],
                sv8.at[2])

        limit = cb + width

        def topk_body(c, carry):
            rk, ri = carry
            seg = c // 8
            off = (c % 8) * L
            kraw = sv8[seg, r8, pl.ds(off, L)]
            gid = iota + (cb + seg * 128 + off)
            vmask = gid < limit
            k = jnp.where(vmask, kraw, jnp.full((L,), -1.0, jnp.float32))
            ids = jnp.minimum(gid, N_ROWS - 1)
            sk, si = _sort16(k, ids, consts)
            return _merge(rk, ri, sk, si, consts)

        rk, ri = lax.fori_loop(
            0, 24, topk_body,
            (jnp.full((L,), NEG_INF, jnp.float32), jnp.zeros((L,), jnp.int32)))
        stage_k[...] = rk
        stage_i[...] = ri

    @pl.when(w >= ACTIVE)
    def _():
        stage_k[...] = jnp.full((L,), NEG_INF, jnp.float32)
        stage_i[...] = jnp.zeros((L,), jnp.int32)

    pltpu.sync_copy(stage_k, ex_k.at[w, :])
    pltpu.sync_copy(stage_i, ex_i.at[w, :])
    plsc.subcore_barrier()

    @pl.when(w == 0)
    def _():
        pltpu.sync_copy(ex_k, all_k)
        pltpu.sync_copy(ex_i, all_i)
        rk = all_k[0, :]
        ri = all_i[0, :]
        for t in range(1, NTILES):
            rk, ri = _merge(rk, ri, all_k[t, :], all_i[t, :], consts)
        stage_k[...] = rk
        pltpu.sync_copy(stage_k, scores_out)
        r8s = []
        copies = []
        for l in range(L):
            rid = _lane(ri, l, consts)
            b8 = pl.multiple_of((rid // 8) * 8, 8)
            r8s.append(rid - b8)
            copies.append(pltpu.async_copy(
                titles_hbm.at[pl.ds(b8, 8), :], t8s.at[l], sem))
        for cp in copies:
            cp.wait()
        for l in range(L):
            for k in range(QV):
                t16v[l, pl.ds(k * L, L)] = t8s[l, r8s[l], pl.ds(k * L, L)]
        pltpu.sync_copy(t16v, titles_out)


@jax.jit
def kernel(movie_title, original_titles, overview_cos_sim):
    mesh = plsc.VectorSubcoreMesh(core_axis_name="c", subcore_axis_name="s",
                                  num_cores=1, num_subcores=NTILES)
    scores16, titles16, _exk, _exi, _exm = pl.kernel(
        _sc_body,
        out_type=(
            jax.ShapeDtypeStruct((L,), jnp.float32),
            jax.ShapeDtypeStruct((L, ROW_LEN), jnp.int32),
            jax.ShapeDtypeStruct((NTILES, L), jnp.float32),
            jax.ShapeDtypeStruct((NTILES, L), jnp.int32),
            jax.ShapeDtypeStruct((NTILES, L), jnp.int32),
        ),
        mesh=mesh,
        scratch_types=[
            pltpu.VMEM((ROW_LEN,), jnp.int32),
            pltpu.VMEM((CHUNK, ROW_LEN), jnp.int32),
            pltpu.VMEM((3, 8, 128), jnp.float32),
            pltpu.VMEM((8, 64), jnp.float32),
            pltpu.VMEM((L,), jnp.float32),
            pltpu.VMEM((L,), jnp.int32),
            pltpu.VMEM((L,), jnp.int32),
            pltpu.VMEM((NTILES, L), jnp.float32),
            pltpu.VMEM((NTILES, L), jnp.int32),
            pltpu.VMEM((NTILES, L), jnp.int32),
            pltpu.VMEM((L, 8, ROW_LEN), jnp.int32),
            pltpu.VMEM((L, ROW_LEN), jnp.int32),
            pltpu.SMEM((8,), jnp.int32),
            pltpu.SemaphoreType.DMA,
        ],
    )(movie_title, original_titles, overview_cos_sim)
    return titles16[1:11], scores16[1:11]


# speedup vs baseline: 3.6744x; 1.0106x over previous
import jax
import jax.numpy as jnp
from jax import lax
from jax.experimental import pallas as pl
from jax.experimental.pallas import tpu as pltpu
from jax.experimental.pallas import tpu_sc as plsc

N_ROWS = 4800
ROW_LEN = 64
L = 16
QV = ROW_LEN // L
NTILES = 16
ACTIVE = 15
CHUNK = N_ROWS // ACTIVE
CVECS = CHUNK // L
NEG_INF = float("-inf")
BIG = 1 << 30

_GDN = lax.GatherDimensionNumbers(
    offset_dims=(), collapsed_slice_dims=(0,), start_index_map=(0,))


def _dg(v, perm):
    return lax.gather(v, perm[:, None], _GDN, (1,),
                      mode=lax.GatherScatterMode.PROMISE_IN_BOUNDS)


def _lane0(v):
    return jnp.reshape(lax.slice(v, (0,), (1,)), ())


def _lane(v, l, consts):
    if l == 0:
        return _lane0(v)
    return _lane0(_dg(v, consts["xorp"][l]))


def _beats(ak, ai, bk, bi):
    one = jnp.ones((L,), jnp.int32)
    zero = jnp.zeros((L,), jnp.int32)
    tie = jnp.where(ai < bi, one, zero)
    return jnp.where(ak > bk, one, jnp.where(ak == bk, tie, zero))


def _cmpx(k, i, perm, keepw):
    pk = _dg(k, perm)
    pi = _dg(i, perm)
    take = _beats(k, i, pk, pi) == keepw
    return jnp.where(take, k, pk), jnp.where(take, i, pi)


def _sort16(k, i, consts):
    for perm, keepw in consts["sort"]:
        k, i = _cmpx(k, i, perm, keepw)
    return k, i


def _cleanup(k, i, consts):
    for perm, keepw in consts["clean"]:
        k, i = _cmpx(k, i, perm, keepw)
    return k, i


def _merge(rk, ri, bk, bi, consts):
    rbk = lax.rev(bk, (0,))
    rbi = lax.rev(bi, (0,))
    win = _beats(rk, ri, rbk, rbi) != 0
    nk = jnp.where(win, rk, rbk)
    ni = jnp.where(win, ri, rbi)
    return _cleanup(nk, ni, consts)


def _or_reduce0(d, consts):
    for p in consts["bfly"]:
        d = d | _dg(d, p)
    return _lane0(d)


def _make_consts():
    iota = lax.iota(jnp.int32, L)
    c = {}
    c["sort"] = []
    for s in range(1, 5):
        for j in range(s - 1, -1, -1):
            upb = (iota >> j) & 1
            descb = (iota >> s) & 1
            c["sort"].append((iota ^ (1 << j), 1 - (upb ^ descb)))
    c["clean"] = [(iota ^ (1 << j), 1 - ((iota >> j) & 1))
                  for j in (3, 2, 1, 0)]
    c["bfly"] = [iota ^ m for m in (1, 2, 4, 8)]
    c["xorp"] = [iota ^ l for l in range(L)]
    return c


def _sc_body(q_hbm, titles_hbm, cos_hbm, scores_out, titles_out,
             ex_k, ex_i, ex_m,
             qv, tv, sv8, svp, stage_k, stage_i, stage_m,
             all_k, all_i, all_m, t8s, t16v, sm, sem):
    w = lax.axis_index("s")
    iota = lax.iota(jnp.int32, L)
    consts = _make_consts()

    @pl.when(w < ACTIVE)
    def _():
        pltpu.sync_copy(q_hbm, qv)
        pltpu.sync_copy(titles_hbm.at[pl.ds(w * CHUNK, CHUNK), :], tv)
        qs = [qv[pl.ds(k * L, L)] for k in range(QV)]

        bigv = jnp.full((L,), BIG, jnp.int32)

        def row_body(i, cand):
            r0 = i * 8
            for u in range(8):
                r = r0 + u
                d = (tv[r, pl.ds(0, L)] ^ qs[0]) | (tv[r, pl.ds(L, L)] ^ qs[1])
                m = d == 0
                cand = jnp.minimum(
                    cand, jnp.where(m, jnp.full((L,), r, jnp.int32), bigv))
            return cand

        cand = lax.fori_loop(0, CHUNK // 8, row_body,
                             jnp.full((L,), BIG, jnp.int32))
        acc = jnp.int32(-1)
        for l in range(L):
            local = _lane(cand, l, consts)
            lc = jnp.clip(local, 0, CHUNK - 1)
            d = tv[lc, pl.ds(0, L)] ^ qs[0]
            for k in range(1, QV):
                d = d | (tv[lc, pl.ds(k * L, L)] ^ qs[k])
            dd = _or_reduce0(d, consts)
            hit = (dd == 0) & (local < CHUNK)
            acc = jnp.where(hit, w * CHUNK + local, acc)
        stage_m[...] = jnp.full((L,), acc, jnp.int32)

    @pl.when(w >= ACTIVE)
    def _():
        stage_m[...] = jnp.full((L,), -1, jnp.int32)

    pltpu.sync_copy(stage_m, ex_m.at[w, :])
    plsc.subcore_barrier()

    pltpu.sync_copy(ex_m, all_m)
    mv = all_m[0, :]
    for t in range(1, NTILES):
        mv = jnp.maximum(mv, all_m[t, :])
    idx0 = _lane0(mv)
    sm[0] = idx0

    @pl.when(idx0 < 0)
    def _():
        @pl.when(w < ACTIVE)
        def _():
            qs = [qv[pl.ds(k * L, L)] for k in range(QV)]

            def row_body(r, acc):
                d = tv[r, pl.ds(0, L)] ^ qs[0]
                for k in range(1, QV):
                    d = d | (tv[r, pl.ds(k * L, L)] ^ qs[k])
                dd = _or_reduce0(d, consts)
                return jnp.where(dd == 0, w * CHUNK + r, acc)

            acc = lax.fori_loop(0, CHUNK, row_body, jnp.int32(-1))
            stage_m[...] = jnp.full((L,), acc, jnp.int32)

        pltpu.sync_copy(stage_m, ex_m.at[w, :])
        plsc.subcore_barrier()
        pltpu.sync_copy(ex_m, all_m)
        mv2 = all_m[0, :]
        for t in range(1, NTILES):
            mv2 = jnp.maximum(mv2, all_m[t, :])
        sm[0] = _lane0(mv2)

    idx = sm[0]
    base8 = pl.multiple_of((idx // 8) * 8, 8)
    r8 = idx - base8

    cb = pl.multiple_of(
        jnp.where(w < 8, 384 * w, 3072 + 256 * (w - 8)), 128)
    width = jnp.where(w < 8, 384, jnp.where(w < 14, 256, 192))

    @pl.when(w < ACTIVE)
    def _():
        pltpu.sync_copy(
            cos_hbm.at[pl.ds(base8, 8), pl.ds(cb, 128)], sv8.at[0])

        @pl.when(w < 14)
        def _():
            pltpu.sync_copy(
                cos_hbm.at[pl.ds(base8, 8),
                           pl.ds(pl.multiple_of(cb + 128, 128), 128)],
                sv8.at[1])

        @pl.when(w == 14)
        def _():
            pltpu.sync_copy(
                cos_hbm.at[pl.ds(base8, 8), pl.ds(4736, 64)], svp)
            for j in range(4):
                sv8[1, r8, pl.ds(j * L, L)] = svp[r8, pl.ds(j * L, L)]

        @pl.when(w < 8)
        def _():
            pltpu.sync_copy(
                cos_hbm.at[pl.ds(base8, 8),
                           pl.ds(pl.multiple_of(cb + 256, 128), 128)],
                sv8.at[2])

        limit = cb + width
        neg1 = jnp.full((L,), -1.0, jnp.float32)

        def load_chunk(c):
            seg = c // 8
            off = (c % 8) * L
            kraw = sv8[seg, r8, pl.ds(off, L)]
            gid = iota + (cb + seg * 128 + off)
            k = jnp.where(gid < limit, kraw, neg1)
            return k, jnp.minimum(gid, N_ROWS - 1)

        def topk_body(p, carry):
            rk, ri = carry
            ka, ia = load_chunk(2 * p)
            kb, ib = load_chunk(2 * p + 1)
            ska, sia = _sort16(ka, ia, consts)
            skb, sib = _sort16(kb, ib, consts)
            mk, mi = _merge(ska, sia, skb, sib, consts)
            return _merge(rk, ri, mk, mi, consts)

        rk, ri = lax.fori_loop(
            0, 12, topk_body,
            (jnp.full((L,), NEG_INF, jnp.float32), jnp.zeros((L,), jnp.int32)))
        stage_k[...] = rk
        stage_i[...] = ri

    @pl.when(w >= ACTIVE)
    def _():
        stage_k[...] = jnp.full((L,), NEG_INF, jnp.float32)
        stage_i[...] = jnp.zeros((L,), jnp.int32)

    pltpu.sync_copy(stage_k, ex_k.at[w, :])
    pltpu.sync_copy(stage_i, ex_i.at[w, :])
    plsc.subcore_barrier()

    @pl.when(w == 0)
    def _():
        pltpu.sync_copy(ex_k, all_k)
        pltpu.sync_copy(ex_i, all_i)
        rk = all_k[0, :]
        ri = all_i[0, :]
        for t in range(1, NTILES):
            rk, ri = _merge(rk, ri, all_k[t, :], all_i[t, :], consts)
        stage_k[...] = rk
        pltpu.sync_copy(stage_k, scores_out)
        r8s = {}
        copies = []
        for l in range(1, 11):
            rid = _lane(ri, l, consts)
            b8 = pl.multiple_of((rid // 8) * 8, 8)
            r8s[l] = rid - b8
            copies.append(pltpu.async_copy(
                titles_hbm.at[pl.ds(b8, 8), :], t8s.at[l], sem))
        for cp in copies:
            cp.wait()
        for l in range(1, 11):
            for k in range(QV):
                t16v[l, pl.ds(k * L, L)] = t8s[l, r8s[l], pl.ds(k * L, L)]
        pltpu.sync_copy(t16v, titles_out)


@jax.jit
def kernel(movie_title, original_titles, overview_cos_sim):
    mesh = plsc.VectorSubcoreMesh(core_axis_name="c", subcore_axis_name="s",
                                  num_cores=1, num_subcores=NTILES)
    scores16, titles16, _exk, _exi, _exm = pl.kernel(
        _sc_body,
        out_type=(
            jax.ShapeDtypeStruct((L,), jnp.float32),
            jax.ShapeDtypeStruct((L, ROW_LEN), jnp.int32),
            jax.ShapeDtypeStruct((NTILES, L), jnp.float32),
            jax.ShapeDtypeStruct((NTILES, L), jnp.int32),
            jax.ShapeDtypeStruct((NTILES, L), jnp.int32),
        ),
        mesh=mesh,
        scratch_types=[
            pltpu.VMEM((ROW_LEN,), jnp.int32),
            pltpu.VMEM((CHUNK, ROW_LEN), jnp.int32),
            pltpu.VMEM((3, 8, 128), jnp.float32),
            pltpu.VMEM((8, 64), jnp.float32),
            pltpu.VMEM((L,), jnp.float32),
            pltpu.VMEM((L,), jnp.int32),
            pltpu.VMEM((L,), jnp.int32),
            pltpu.VMEM((NTILES, L), jnp.float32),
            pltpu.VMEM((NTILES, L), jnp.int32),
            pltpu.VMEM((NTILES, L), jnp.int32),
            pltpu.VMEM((L, 8, ROW_LEN), jnp.int32),
            pltpu.VMEM((L, ROW_LEN), jnp.int32),
            pltpu.SMEM((8,), jnp.int32),
            pltpu.SemaphoreType.DMA,
        ],
    )(movie_title, original_titles, overview_cos_sim)
    return titles16[1:11], scores16[1:11]


# speedup vs baseline: 3.8191x; 1.0394x over previous
import jax
import jax.numpy as jnp
from jax import lax
from jax.experimental import pallas as pl
from jax.experimental.pallas import tpu as pltpu
from jax.experimental.pallas import tpu_sc as plsc

N_ROWS = 4800
ROW_LEN = 64
L = 16
QV = ROW_LEN // L
NTILES = 16
ACTIVE = 15
CHUNK = N_ROWS // ACTIVE
CVECS = CHUNK // L
NEG_INF = float("-inf")
BIG = 1 << 30

_GDN = lax.GatherDimensionNumbers(
    offset_dims=(), collapsed_slice_dims=(0,), start_index_map=(0,))


def _dg(v, perm):
    return lax.gather(v, perm[:, None], _GDN, (1,),
                      mode=lax.GatherScatterMode.PROMISE_IN_BOUNDS)


def _lane0(v):
    return jnp.reshape(lax.slice(v, (0,), (1,)), ())


def _lane(v, l, consts):
    if l == 0:
        return _lane0(v)
    return _lane0(_dg(v, consts["xorp"][l]))


def _beats(ak, ai, bk, bi):
    one = jnp.ones((L,), jnp.int32)
    zero = jnp.zeros((L,), jnp.int32)
    tie = jnp.where(ai < bi, one, zero)
    return jnp.where(ak > bk, one, jnp.where(ak == bk, tie, zero))


def _cmpx(k, i, perm, keepw):
    pk = _dg(k, perm)
    pi = _dg(i, perm)
    take = _beats(k, i, pk, pi) == keepw
    return jnp.where(take, k, pk), jnp.where(take, i, pi)


def _sort16(k, i, consts):
    for perm, keepw in consts["sort"]:
        k, i = _cmpx(k, i, perm, keepw)
    return k, i


def _cleanup(k, i, consts):
    for perm, keepw in consts["clean"]:
        k, i = _cmpx(k, i, perm, keepw)
    return k, i


def _merge(rk, ri, bk, bi, consts):
    rbk = lax.rev(bk, (0,))
    rbi = lax.rev(bi, (0,))
    win = _beats(rk, ri, rbk, rbi) != 0
    nk = jnp.where(win, rk, rbk)
    ni = jnp.where(win, ri, rbi)
    return _cleanup(nk, ni, consts)


def _or_reduce0(d, consts):
    for p in consts["bfly"]:
        d = d | _dg(d, p)
    return _lane0(d)


def _make_consts():
    iota = lax.iota(jnp.int32, L)
    c = {}
    c["sort"] = []
    for s in range(1, 5):
        for j in range(s - 1, -1, -1):
            upb = (iota >> j) & 1
            descb = (iota >> s) & 1
            c["sort"].append((iota ^ (1 << j), 1 - (upb ^ descb)))
    c["clean"] = [(iota ^ (1 << j), 1 - ((iota >> j) & 1))
                  for j in (3, 2, 1, 0)]
    c["bfly"] = [iota ^ m for m in (1, 2, 4, 8)]
    c["xorp"] = [iota ^ l for l in range(L)]
    return c


def _sc_body(q_hbm, titles_hbm, cos_hbm, scores_out, titles_out,
             ex_k, ex_i, ex_m,
             qv, tv, sv8, svp, stage_k, stage_i, stage_m,
             all_k, all_i, all_m, t16v, sm, sem):
    w = lax.axis_index("s")
    iota = lax.iota(jnp.int32, L)
    consts = _make_consts()

    @pl.when(w < ACTIVE)
    def _():
        pltpu.sync_copy(q_hbm, qv)
        pltpu.sync_copy(
            titles_hbm.at[pl.ds(w * CHUNK * ROW_LEN, CHUNK * ROW_LEN)], tv)
        qs = [qv[pl.ds(k * L, L)] for k in range(QV)]

        bigv = jnp.full((L,), BIG, jnp.int32)

        def row_body(i, cand):
            r0 = i * 8
            for u in range(8):
                r = r0 + u
                b = r * ROW_LEN
                d = ((tv[pl.ds(b, L)] ^ qs[0])
                     | (tv[pl.ds(b + L, L)] ^ qs[1]))
                m = d == 0
                cand = jnp.minimum(
                    cand, jnp.where(m, jnp.full((L,), r, jnp.int32), bigv))
            return cand

        cand = lax.fori_loop(0, CHUNK // 8, row_body,
                             jnp.full((L,), BIG, jnp.int32))
        acc = jnp.int32(-1)
        for l in range(L):
            local = _lane(cand, l, consts)
            lc = jnp.clip(local, 0, CHUNK - 1) * ROW_LEN
            d = tv[pl.ds(lc, L)] ^ qs[0]
            for k in range(1, QV):
                d = d | (tv[pl.ds(lc + k * L, L)] ^ qs[k])
            dd = _or_reduce0(d, consts)
            hit = (dd == 0) & (local < CHUNK)
            acc = jnp.where(hit, w * CHUNK + local, acc)
        stage_m[...] = jnp.full((L,), acc, jnp.int32)

    @pl.when(w >= ACTIVE)
    def _():
        stage_m[...] = jnp.full((L,), -1, jnp.int32)

    pltpu.sync_copy(stage_m, ex_m.at[w, :])
    plsc.subcore_barrier()

    pltpu.sync_copy(ex_m, all_m)
    mv = all_m[0, :]
    for t in range(1, NTILES):
        mv = jnp.maximum(mv, all_m[t, :])
    idx0 = _lane0(mv)
    sm[0] = idx0

    @pl.when(idx0 < 0)
    def _():
        @pl.when(w < ACTIVE)
        def _():
            qs = [qv[pl.ds(k * L, L)] for k in range(QV)]

            def row_body(r, acc):
                b = r * ROW_LEN
                d = tv[pl.ds(b, L)] ^ qs[0]
                for k in range(1, QV):
                    d = d | (tv[pl.ds(b + k * L, L)] ^ qs[k])
                dd = _or_reduce0(d, consts)
                return jnp.where(dd == 0, w * CHUNK + r, acc)

            acc = lax.fori_loop(0, CHUNK, row_body, jnp.int32(-1))
            stage_m[...] = jnp.full((L,), acc, jnp.int32)

        pltpu.sync_copy(stage_m, ex_m.at[w, :])
        plsc.subcore_barrier()
        pltpu.sync_copy(ex_m, all_m)
        mv2 = all_m[0, :]
        for t in range(1, NTILES):
            mv2 = jnp.maximum(mv2, all_m[t, :])
        sm[0] = _lane0(mv2)

    idx = sm[0]
    base8 = pl.multiple_of((idx // 8) * 8, 8)
    r8 = idx - base8

    cb = pl.multiple_of(
        jnp.where(w < 8, 384 * w, 3072 + 256 * (w - 8)), 128)
    width = jnp.where(w < 8, 384, jnp.where(w < 14, 256, 192))

    @pl.when(w < ACTIVE)
    def _():
        pltpu.sync_copy(
            cos_hbm.at[pl.ds(base8, 8), pl.ds(cb, 128)], sv8.at[0])

        @pl.when(w < 14)
        def _():
            pltpu.sync_copy(
                cos_hbm.at[pl.ds(base8, 8),
                           pl.ds(pl.multiple_of(cb + 128, 128), 128)],
                sv8.at[1])

        @pl.when(w == 14)
        def _():
            pltpu.sync_copy(
                cos_hbm.at[pl.ds(base8, 8), pl.ds(4736, 64)], svp)
            for j in range(4):
                sv8[1, r8, pl.ds(j * L, L)] = svp[r8, pl.ds(j * L, L)]

        @pl.when(w < 8)
        def _():
            pltpu.sync_copy(
                cos_hbm.at[pl.ds(base8, 8),
                           pl.ds(pl.multiple_of(cb + 256, 128), 128)],
                sv8.at[2])

        limit = cb + width
        neg1 = jnp.full((L,), -1.0, jnp.float32)

        def load_chunk(c):
            seg = c // 8
            off = (c % 8) * L
            kraw = sv8[seg, r8, pl.ds(off, L)]
            gid = iota + (cb + seg * 128 + off)
            k = jnp.where(gid < limit, kraw, neg1)
            return k, jnp.minimum(gid, N_ROWS - 1)

        def topk_body(p, carry):
            rk, ri = carry
            ka, ia = load_chunk(2 * p)
            kb, ib = load_chunk(2 * p + 1)
            ska, sia = _sort16(ka, ia, consts)
            skb, sib = _sort16(kb, ib, consts)
            mk, mi = _merge(ska, sia, skb, sib, consts)
            return _merge(rk, ri, mk, mi, consts)

        rk, ri = lax.fori_loop(
            0, 12, topk_body,
            (jnp.full((L,), NEG_INF, jnp.float32), jnp.zeros((L,), jnp.int32)))
        stage_k[...] = rk
        stage_i[...] = ri

    @pl.when(w >= ACTIVE)
    def _():
        stage_k[...] = jnp.full((L,), NEG_INF, jnp.float32)
        stage_i[...] = jnp.zeros((L,), jnp.int32)

    pltpu.sync_copy(stage_k, ex_k.at[w, :])
    pltpu.sync_copy(stage_i, ex_i.at[w, :])
    plsc.subcore_barrier()

    @pl.when(w == 0)
    def _():
        pltpu.sync_copy(ex_k, all_k)
        pltpu.sync_copy(ex_i, all_i)
        rk = all_k[0, :]
        ri = all_i[0, :]
        for t in range(1, NTILES):
            rk, ri = _merge(rk, ri, all_k[t, :], all_i[t, :], consts)
        stage_k[...] = rk
        pltpu.sync_copy(stage_k, scores_out)
        copies = []
        for l in range(1, 11):
            rid = _lane(ri, l, consts)
            copies.append(pltpu.async_copy(
                titles_hbm.at[pl.ds(pl.multiple_of(rid * ROW_LEN, 8),
                                    ROW_LEN)],
                t16v.at[l, :], sem))
        for cp in copies:
            cp.wait()
        pltpu.sync_copy(t16v, titles_out)


@jax.jit
def kernel(movie_title, original_titles, overview_cos_sim):
    titles_flat = original_titles.reshape(N_ROWS * ROW_LEN)
    mesh = plsc.VectorSubcoreMesh(core_axis_name="c", subcore_axis_name="s",
                                  num_cores=1, num_subcores=NTILES)
    scores16, titles16, _exk, _exi, _exm = pl.kernel(
        _sc_body,
        out_type=(
            jax.ShapeDtypeStruct((L,), jnp.float32),
            jax.ShapeDtypeStruct((L, ROW_LEN), jnp.int32),
            jax.ShapeDtypeStruct((NTILES, L), jnp.float32),
            jax.ShapeDtypeStruct((NTILES, L), jnp.int32),
            jax.ShapeDtypeStruct((NTILES, L), jnp.int32),
        ),
        mesh=mesh,
        scratch_types=[
            pltpu.VMEM((ROW_LEN,), jnp.int32),
            pltpu.VMEM((CHUNK * ROW_LEN,), jnp.int32),
            pltpu.VMEM((3, 8, 128), jnp.float32),
            pltpu.VMEM((8, 64), jnp.float32),
            pltpu.VMEM((L,), jnp.float32),
            pltpu.VMEM((L,), jnp.int32),
            pltpu.VMEM((L,), jnp.int32),
            pltpu.VMEM((NTILES, L), jnp.float32),
            pltpu.VMEM((NTILES, L), jnp.int32),
            pltpu.VMEM((NTILES, L), jnp.int32),
            pltpu.VMEM((L, ROW_LEN), jnp.int32),
            pltpu.SMEM((8,), jnp.int32),
            pltpu.SemaphoreType.DMA,
        ],
    )(movie_title, titles_flat, overview_cos_sim)
    return titles16[1:11], scores16[1:11]


# speedup vs baseline: 4.0424x; 1.0585x over previous
import jax
import jax.numpy as jnp
from jax import lax
from jax.experimental import pallas as pl
from jax.experimental.pallas import tpu as pltpu
from jax.experimental.pallas import tpu_sc as plsc

N_ROWS = 4800
ROW_LEN = 64
L = 16
QV = ROW_LEN // L
NTILES = 16
ACTIVE = 15
CHUNK = N_ROWS // ACTIVE
CVECS = CHUNK // L
NEG_INF = float("-inf")
BIG = 1 << 30

_GDN = lax.GatherDimensionNumbers(
    offset_dims=(), collapsed_slice_dims=(0,), start_index_map=(0,))


def _dg(v, perm):
    return lax.gather(v, perm[:, None], _GDN, (1,),
                      mode=lax.GatherScatterMode.PROMISE_IN_BOUNDS)


def _lane0(v):
    return jnp.reshape(lax.slice(v, (0,), (1,)), ())


def _lane(v, l, consts):
    if l == 0:
        return _lane0(v)
    return _lane0(_dg(v, consts["xorp"][l]))


def _beats(ak, ai, bk, bi):
    one = jnp.ones((L,), jnp.int32)
    zero = jnp.zeros((L,), jnp.int32)
    tie = jnp.where(ai < bi, one, zero)
    return jnp.where(ak > bk, one, jnp.where(ak == bk, tie, zero))


def _cmpx(k, i, perm, keepw):
    pk = _dg(k, perm)
    pi = _dg(i, perm)
    take = _beats(k, i, pk, pi) == keepw
    return jnp.where(take, k, pk), jnp.where(take, i, pi)


def _sort16(k, i, consts):
    for perm, keepw in consts["sort"]:
        k, i = _cmpx(k, i, perm, keepw)
    return k, i


def _cleanup(k, i, consts):
    for perm, keepw in consts["clean"]:
        k, i = _cmpx(k, i, perm, keepw)
    return k, i


def _merge(rk, ri, bk, bi, consts):
    rbk = lax.rev(bk, (0,))
    rbi = lax.rev(bi, (0,))
    win = _beats(rk, ri, rbk, rbi) != 0
    nk = jnp.where(win, rk, rbk)
    ni = jnp.where(win, ri, rbi)
    return _cleanup(nk, ni, consts)


def _or_reduce0(d, consts):
    for p in consts["bfly"]:
        d = d | _dg(d, p)
    return _lane0(d)


def _make_consts():
    iota = lax.iota(jnp.int32, L)
    c = {}
    c["sort"] = []
    for s in range(1, 5):
        for j in range(s - 1, -1, -1):
            upb = (iota >> j) & 1
            descb = (iota >> s) & 1
            c["sort"].append((iota ^ (1 << j), 1 - (upb ^ descb)))
    c["clean"] = [(iota ^ (1 << j), 1 - ((iota >> j) & 1))
                  for j in (3, 2, 1, 0)]
    c["bfly"] = [iota ^ m for m in (1, 2, 4, 8)]
    c["xorp"] = [iota ^ l for l in range(L)]
    return c


def _sc_body(q_hbm, titles_hbm, cos_hbm, scores_out, titles_out,
             ex_k, ex_i, ex_m,
             qv, tv, sv8, svp, stage_k, stage_i, stage_m,
             all_k, all_i, all_m, t16v, sm, sem):
    w = lax.axis_index("s")
    iota = lax.iota(jnp.int32, L)
    consts = _make_consts()

    @pl.when(w < ACTIVE)
    def _():
        pltpu.sync_copy(q_hbm, qv)
        pltpu.sync_copy(
            titles_hbm.at[pl.ds(w * CHUNK * ROW_LEN, CHUNK * ROW_LEN)], tv)
        qs = [qv[pl.ds(k * L, L)] for k in range(QV)]

        bigv = jnp.full((L,), BIG, jnp.int32)

        def row_body(i, cands):
            r0 = i * 8
            out = []
            for u in range(8):
                r = r0 + u
                b = r * ROW_LEN
                d = ((tv[pl.ds(b, L)] ^ qs[0])
                     | (tv[pl.ds(b + L, L)] ^ qs[1]))
                m = d == 0
                out.append(jnp.minimum(
                    cands[u],
                    jnp.where(m, jnp.full((L,), r, jnp.int32), bigv)))
            return tuple(out)

        cands = lax.fori_loop(0, CHUNK // 8, row_body, (bigv,) * 8)
        cand = cands[0]
        for u in range(1, 8):
            cand = jnp.minimum(cand, cands[u])
        for p in consts["bfly"]:
            cand = jnp.minimum(cand, _dg(cand, p))
        local = _lane0(cand)
        lc = jnp.clip(local, 0, CHUNK - 1) * ROW_LEN
        d = tv[pl.ds(lc, L)] ^ qs[0]
        for k in range(1, QV):
            d = d | (tv[pl.ds(lc + k * L, L)] ^ qs[k])
        dd = _or_reduce0(d, consts)
        hit = (dd == 0) & (local < CHUNK)
        acc = jnp.where(hit, w * CHUNK + local, jnp.int32(-1))
        stage_m[...] = jnp.full((L,), acc, jnp.int32)

    @pl.when(w >= ACTIVE)
    def _():
        stage_m[...] = jnp.full((L,), -1, jnp.int32)

    pltpu.sync_copy(stage_m, ex_m.at[w, :])
    plsc.subcore_barrier()

    pltpu.sync_copy(ex_m, all_m)
    mv = all_m[0, :]
    for t in range(1, NTILES):
        mv = jnp.maximum(mv, all_m[t, :])
    idx0 = _lane0(mv)
    sm[0] = idx0

    @pl.when(idx0 < 0)
    def _():
        @pl.when(w < ACTIVE)
        def _():
            qs = [qv[pl.ds(k * L, L)] for k in range(QV)]

            def row_body(r, acc):
                b = r * ROW_LEN
                d = tv[pl.ds(b, L)] ^ qs[0]
                for k in range(1, QV):
                    d = d | (tv[pl.ds(b + k * L, L)] ^ qs[k])
                dd = _or_reduce0(d, consts)
                return jnp.where(dd == 0, w * CHUNK + r, acc)

            acc = lax.fori_loop(0, CHUNK, row_body, jnp.int32(-1))
            stage_m[...] = jnp.full((L,), acc, jnp.int32)

        pltpu.sync_copy(stage_m, ex_m.at[w, :])
        plsc.subcore_barrier()
        pltpu.sync_copy(ex_m, all_m)
        mv2 = all_m[0, :]
        for t in range(1, NTILES):
            mv2 = jnp.maximum(mv2, all_m[t, :])
        sm[0] = _lane0(mv2)

    idx = sm[0]
    base8 = pl.multiple_of((idx // 8) * 8, 8)
    r8 = idx - base8

    cb = pl.multiple_of(
        jnp.where(w < 8, 384 * w, 3072 + 256 * (w - 8)), 128)
    width = jnp.where(w < 8, 384, jnp.where(w < 14, 256, 192))

    @pl.when(w < ACTIVE)
    def _():
        pltpu.sync_copy(
            cos_hbm.at[pl.ds(base8, 8), pl.ds(cb, 128)], sv8.at[0])

        @pl.when(w < 14)
        def _():
            pltpu.sync_copy(
                cos_hbm.at[pl.ds(base8, 8),
                           pl.ds(pl.multiple_of(cb + 128, 128), 128)],
                sv8.at[1])

        @pl.when(w == 14)
        def _():
            pltpu.sync_copy(
                cos_hbm.at[pl.ds(base8, 8), pl.ds(4736, 64)], svp)
            for j in range(4):
                sv8[1, r8, pl.ds(j * L, L)] = svp[r8, pl.ds(j * L, L)]

        @pl.when(w < 8)
        def _():
            pltpu.sync_copy(
                cos_hbm.at[pl.ds(base8, 8),
                           pl.ds(pl.multiple_of(cb + 256, 128), 128)],
                sv8.at[2])

        limit = cb + width
        neg1 = jnp.full((L,), -1.0, jnp.float32)

        def load_chunk(c):
            seg = c // 8
            off = (c % 8) * L
            kraw = sv8[seg, r8, pl.ds(off, L)]
            gid = iota + (cb + seg * 128 + off)
            k = jnp.where(gid < limit, kraw, neg1)
            return k, jnp.minimum(gid, N_ROWS - 1)

        def topk_body(p, carry):
            rk, ri = carry
            ka, ia = load_chunk(2 * p)
            kb, ib = load_chunk(2 * p + 1)
            ska, sia = _sort16(ka, ia, consts)
            skb, sib = _sort16(kb, ib, consts)
            mk, mi = _merge(ska, sia, skb, sib, consts)
            return _merge(rk, ri, mk, mi, consts)

        rk, ri = lax.fori_loop(
            0, 12, topk_body,
            (jnp.full((L,), NEG_INF, jnp.float32), jnp.zeros((L,), jnp.int32)))
        stage_k[...] = rk
        stage_i[...] = ri

    @pl.when(w >= ACTIVE)
    def _():
        stage_k[...] = jnp.full((L,), NEG_INF, jnp.float32)
        stage_i[...] = jnp.zeros((L,), jnp.int32)

    pltpu.sync_copy(stage_k, ex_k.at[w, :])
    pltpu.sync_copy(stage_i, ex_i.at[w, :])
    plsc.subcore_barrier()

    @pl.when(w == 0)
    def _():
        pltpu.sync_copy(ex_k, all_k)
        pltpu.sync_copy(ex_i, all_i)
        rk = all_k[0, :]
        ri = all_i[0, :]
        for t in range(1, NTILES):
            rk, ri = _merge(rk, ri, all_k[t, :], all_i[t, :], consts)
        stage_k[...] = rk
        pltpu.sync_copy(stage_k, scores_out)
        copies = []
        for l in range(1, 11):
            rid = _lane(ri, l, consts)
            copies.append(pltpu.async_copy(
                titles_hbm.at[pl.ds(pl.multiple_of(rid * ROW_LEN, 8),
                                    ROW_LEN)],
                t16v.at[l, :], sem))
        for cp in copies:
            cp.wait()
        pltpu.sync_copy(t16v, titles_out)


@jax.jit
def kernel(movie_title, original_titles, overview_cos_sim):
    titles_flat = original_titles.reshape(N_ROWS * ROW_LEN)
    mesh = plsc.VectorSubcoreMesh(core_axis_name="c", subcore_axis_name="s",
                                  num_cores=1, num_subcores=NTILES)
    scores16, titles16, _exk, _exi, _exm = pl.kernel(
        _sc_body,
        out_type=(
            jax.ShapeDtypeStruct((L,), jnp.float32),
            jax.ShapeDtypeStruct((L, ROW_LEN), jnp.int32),
            jax.ShapeDtypeStruct((NTILES, L), jnp.float32),
            jax.ShapeDtypeStruct((NTILES, L), jnp.int32),
            jax.ShapeDtypeStruct((NTILES, L), jnp.int32),
        ),
        mesh=mesh,
        scratch_types=[
            pltpu.VMEM((ROW_LEN,), jnp.int32),
            pltpu.VMEM((CHUNK * ROW_LEN,), jnp.int32),
            pltpu.VMEM((3, 8, 128), jnp.float32),
            pltpu.VMEM((8, 64), jnp.float32),
            pltpu.VMEM((L,), jnp.float32),
            pltpu.VMEM((L,), jnp.int32),
            pltpu.VMEM((L,), jnp.int32),
            pltpu.VMEM((NTILES, L), jnp.float32),
            pltpu.VMEM((NTILES, L), jnp.int32),
            pltpu.VMEM((NTILES, L), jnp.int32),
            pltpu.VMEM((L, ROW_LEN), jnp.int32),
            pltpu.SMEM((8,), jnp.int32),
            pltpu.SemaphoreType.DMA,
        ],
    )(movie_title, titles_flat, overview_cos_sim)
    return titles16[1:11], scores16[1:11]
